# baseline (speedup 1.0000x reference)
import functools

import jax
import jax.numpy as jnp
from jax import lax
from jax.experimental import pallas as pl
from jax.experimental.pallas import tpu as pltpu
from jax.experimental.pallas import tpu_sc as plsc

LP = 112
_L = 16


def _prep_kernel(energy_ref, istrack_ref, trackpt_ref, eta_ref,
                 phi_ref, ismuon_ref, layer_ref, ct_ref):
    energy = energy_ref[0]
    isTrack = istrack_ref[0]
    track_pt = trackpt_ref[0]
    eta = eta_ref[0]
    phi = phi_ref[0]
    isMuon = ismuon_ref[0]
    layer = layer_ref[0]

    nt = (isTrack != 1.0).astype(jnp.float32)
    ne = jnp.exp(energy + 1.0) * nt + isTrack * 1e-8
    ct_ref[0] = jnp.concatenate(
        [
            isTrack * track_pt,
            isTrack * eta,
            isTrack * phi,
            isTrack * isMuon,
            ne * (eta * 1.5),
            ne * (phi * 1.8),
            jnp.exp(energy + 2.0) * nt,
            ne * layer,
            ne,
        ],
        axis=0,
    )


def _sc_agg_body(N, P, inc_hbm, ct_hbm, out_hbm, inc_v, ct_v, out_v):
    c = lax.axis_index("c")
    s = lax.axis_index("s")
    b = s * 2 + c
    npb = N * P

    pltpu.sync_copy(inc_hbm.at[pl.ds(b * npb, npb)], inc_v.at[pl.ds(0, npb)])
    pltpu.sync_copy(ct_hbm.at[b], ct_v)

    tail_mask = lax.iota(jnp.int32, _L) < (P - (P // _L) * _L)
    nph = P // _L + 1
    nchunks = N // _L
    ntail = N - nchunks * _L

    def make_pass(ph0, ph1):
        nphases = ph1 - ph0

        def accumulate(accs, nbase, j0, j1):
            cks = [ct_v[k, pl.ds(nbase, _L)] for k in range(9)]
            accs = list(accs)
            for j in range(j0, j1):
                base = (nbase + j) * P
                for ph in range(ph0, ph1):
                    x = inc_v[pl.ds(base + ph * _L, _L)]
                    if ph == nph - 1:
                        x = jnp.where(tail_mask, x, 0.0)
                    for k in range(9):
                        i = k * nphases + (ph - ph0)
                        accs[i] = accs[i] + x * cks[k][j]
            return tuple(accs)

        def body(i, accs):
            return accumulate(accs, i * _L, 0, _L)

        z = tuple(jnp.zeros((_L,), jnp.float32) for _ in range(9 * nphases))
        accs = lax.fori_loop(0, nchunks, body, z)
        if ntail:
            accs = accumulate(accs, N - _L, _L - ntail, _L)
        for k in range(9):
            for ph in range(ph0, ph1):
                out_v[k, pl.ds(ph * _L, _L)] = accs[k * nphases + (ph - ph0)]

    make_pass(0, 4)
    make_pass(4, nph)
    pltpu.sync_copy(out_v, out_hbm.at[b])


def _heads_kernel(s_ref, feat_ref,
                  w1pa_ref, w1pb_ref, b1p_ref, w2p_ref, b2p_ref, w3p_ref, b3p_ref,
                  w1ca_ref, w1cb_ref, b1c_ref, w2c_ref, b2c_ref, w3c_ref, b3c_ref,
                  outp_ref, outc_ref, topo_ref):
    s = s_ref[...]
    BS, _, _ = s.shape
    P = topo_ref.shape[0] // BS

    denom = s[:, 8:9, :]
    eta_s = s[:, 4:5, :] / denom
    phi_s = s[:, 5:6, :] / denom
    layer_s = s[:, 7:8, :] / denom
    energy_s = s[:, 6:7, :]
    cosh = 0.5 * (jnp.exp(eta_s) + jnp.exp(-eta_s))
    pt = jnp.log(energy_s / cosh) - 2.0
    out8 = jnp.concatenate(
        [s[:, 0:4, :], pt, eta_s / 1.5, phi_s / 1.8, layer_s], axis=1
    )
    t = out8[:, :, 0:P].transpose(0, 2, 1).reshape(BS * P, 8)
    skip = t[:, 0:4]
    topo_ref[...] = t[:, 4:8]

    x = feat_ref[...]
    h = jax.nn.relu(x @ w1pa_ref[...] + skip @ w1pb_ref[...] + b1p_ref[...])
    h = jax.nn.relu(h @ w2p_ref[...] + b2p_ref[...])
    outp_ref[...] = h @ w3p_ref[...] + b3p_ref[...]

    h = jax.nn.relu(x @ w1ca_ref[...] + skip @ w1cb_ref[...] + b1c_ref[...])
    h = jax.nn.relu(h @ w2c_ref[...] + b2c_ref[...])
    o = h @ w3c_ref[...] + b3c_ref[...]
    m = jnp.max(o, axis=1, keepdims=True)
    e = jnp.exp(o - m)
    outc_ref[...] = e / jnp.sum(e, axis=1, keepdims=True)


def kernel(features, energy, isTrack, track_pt, eta, phi, isMuon, layer,
           incidence_val, W1p, b1p, W2p, b2p, W3p, b3p, W1c, b1c, W2c, b2c,
           W3c, b3c, edge_src, edge_dst):
    E = incidence_val.shape[0]
    BSN = energy.shape[0]
    BSP, DIM = features.shape
    P = E // BSN
    BS = BSP // P
    N = BSN // BS

    node3 = lambda a: a.reshape(BS, 1, N)
    nvec = pl.BlockSpec((1, 1, N), lambda b: (b, 0, 0))
    ct = pl.pallas_call(
        _prep_kernel,
        grid=(BS,),
        in_specs=[nvec] * 7,
        out_specs=pl.BlockSpec((1, 9, N), lambda b: (b, 0, 0)),
        out_shape=jax.ShapeDtypeStruct((BS, 9, N), jnp.float32),
    )(node3(energy), node3(isTrack), node3(track_pt), node3(eta),
      node3(phi), node3(isMuon), node3(layer))

    mesh = plsc.VectorSubcoreMesh(core_axis_name="c", subcore_axis_name="s")
    sc_agg = functools.partial(
        pl.kernel,
        out_type=jax.ShapeDtypeStruct((BS, 9, LP), jnp.float32),
        mesh=mesh,
        scratch_types=[
            pltpu.VMEM((N * P + _L,), jnp.float32),
            pltpu.VMEM((9, N), jnp.float32),
            pltpu.VMEM((9, LP), jnp.float32),
        ],
    )(functools.partial(_sc_agg_body, N, P))
    s_agg = sc_agg(incidence_val, ct)

    row2 = lambda a: a.reshape(1, -1)
    hargs = [s_agg, features,
             W1p[:DIM], W1p[DIM:], row2(b1p), W2p, row2(b2p), W3p, row2(b3p),
             W1c[:DIM], W1c[DIM:], row2(b1c), W2c, row2(b2c), W3c, row2(b3c)]
    outp, outc, topo = pl.pallas_call(
        _heads_kernel,
        in_specs=[pl.BlockSpec(a.shape, lambda nd=a.ndim: (0,) * nd)
                  for a in hargs],
        out_specs=[
            pl.BlockSpec((BSP, 3), lambda: (0, 0)),
            pl.BlockSpec((BSP, 6), lambda: (0, 0)),
            pl.BlockSpec((BSP, 4), lambda: (0, 0)),
        ],
        out_shape=[
            jax.ShapeDtypeStruct((BSP, 3), jnp.float32),
            jax.ShapeDtypeStruct((BSP, 6), jnp.float32),
            jax.ShapeDtypeStruct((BSP, 4), jnp.float32),
        ],
    )(*hargs)

    return (outp.reshape(BS, P, 3), outc.reshape(BS, P, 6), topo)


# speedup vs baseline: 1.2980x; 1.2980x over previous
import functools

import jax
import jax.numpy as jnp
from jax import lax
from jax.experimental import pallas as pl
from jax.experimental.pallas import tpu as pltpu
from jax.experimental.pallas import tpu_sc as plsc

LP = 112
_L = 16


def _sc_agg_body(N, P, energy_hbm, istrack_hbm, trackpt_hbm, eta_hbm,
                 phi_hbm, ismuon_hbm, layer_hbm, inc_hbm, out_hbm,
                 inc_v, nd_v, ct_v, out_v):
    c = lax.axis_index("c")
    s = lax.axis_index("s")
    b = s * 2 + c
    npb = N * P

    pltpu.sync_copy(inc_hbm.at[pl.ds(b * npb, npb)], inc_v.at[pl.ds(0, npb)])
    nb = pl.ds(b * N, N)
    for i, a in enumerate([energy_hbm, istrack_hbm, trackpt_hbm, eta_hbm,
                           phi_hbm, ismuon_hbm, layer_hbm]):
        pltpu.sync_copy(a.at[nb], nd_v.at[pl.ds(i * N, N)])

    def build_ct_at(nbase):
        e = nd_v[pl.ds(0 * N + nbase, _L)]
        isTrack = nd_v[pl.ds(1 * N + nbase, _L)]
        track_pt = nd_v[pl.ds(2 * N + nbase, _L)]
        eta = nd_v[pl.ds(3 * N + nbase, _L)]
        phi = nd_v[pl.ds(4 * N + nbase, _L)]
        isMuon = nd_v[pl.ds(5 * N + nbase, _L)]
        layer = nd_v[pl.ds(6 * N + nbase, _L)]
        nt = 1.0 - isTrack
        ne = jnp.exp(e + 1.0) * nt + isTrack * 1e-8
        vals = [isTrack * track_pt, isTrack * eta, isTrack * phi,
                isTrack * isMuon, ne * (eta * 1.5), ne * (phi * 1.8),
                jnp.exp(e + 2.0) * nt, ne * layer, ne]
        for k, v in enumerate(vals):
            ct_v[pl.ds(k * N + nbase, _L)] = v

    nchunks0 = N // _L
    lax.fori_loop(0, nchunks0, lambda i, _: (build_ct_at(i * _L), 0)[1], 0)
    if N - nchunks0 * _L:
        build_ct_at(N - _L)

    nph = P // _L + 1
    nchunks = N // _L
    ntail = N - nchunks * _L

    def make_pass(ph0, ph1):
        nphases = ph1 - ph0

        def accumulate(accs, nbase, j0, j1):
            cks = [ct_v[pl.ds(k * N + nbase, _L)] for k in range(9)]
            accs = list(accs)
            for j in range(j0, j1):
                base = (nbase + j) * P
                for ph in range(ph0, ph1):
                    x = inc_v[pl.ds(base + ph * _L, _L)]
                    for k in range(9):
                        i = k * nphases + (ph - ph0)
                        accs[i] = accs[i] + x * cks[k][j]
            return tuple(accs)

        def body(i, accs):
            return accumulate(accs, i * _L, 0, _L)

        z = tuple(jnp.zeros((_L,), jnp.float32) for _ in range(9 * nphases))
        accs = lax.fori_loop(0, nchunks, body, z)
        if ntail:
            accs = accumulate(accs, N - _L, _L - ntail, _L)
        for k in range(9):
            for ph in range(ph0, ph1):
                out_v[k, pl.ds(ph * _L, _L)] = accs[k * nphases + (ph - ph0)]

    make_pass(0, 4)
    make_pass(4, nph)
    pltpu.sync_copy(out_v, out_hbm.at[b])


def _heads_kernel(s_ref, feat_ref,
                  w1pa_ref, w1pb_ref, b1p_ref, w2p_ref, b2p_ref, w3p_ref, b3p_ref,
                  w1ca_ref, w1cb_ref, b1c_ref, w2c_ref, b2c_ref, w3c_ref, b3c_ref,
                  outp_ref, outc_ref, topo_ref):
    s = s_ref[...]
    BS, _, _ = s.shape
    P = topo_ref.shape[0] // BS

    denom = s[:, 8:9, :]
    eta_s = s[:, 4:5, :] / denom
    phi_s = s[:, 5:6, :] / denom
    layer_s = s[:, 7:8, :] / denom
    energy_s = s[:, 6:7, :]
    cosh = 0.5 * (jnp.exp(eta_s) + jnp.exp(-eta_s))
    pt = jnp.log(energy_s / cosh) - 2.0
    out8 = jnp.concatenate(
        [s[:, 0:4, :], pt, eta_s / 1.5, phi_s / 1.8, layer_s], axis=1
    )
    t = out8[:, :, 0:P].transpose(0, 2, 1).reshape(BS * P, 8)
    skip = t[:, 0:4]
    topo_ref[...] = t[:, 4:8]

    x = feat_ref[...]
    h = jax.nn.relu(x @ w1pa_ref[...] + skip @ w1pb_ref[...] + b1p_ref[...])
    h = jax.nn.relu(h @ w2p_ref[...] + b2p_ref[...])
    outp_ref[...] = h @ w3p_ref[...] + b3p_ref[...]

    h = jax.nn.relu(x @ w1ca_ref[...] + skip @ w1cb_ref[...] + b1c_ref[...])
    h = jax.nn.relu(h @ w2c_ref[...] + b2c_ref[...])
    o = h @ w3c_ref[...] + b3c_ref[...]
    m = jnp.max(o, axis=1, keepdims=True)
    e = jnp.exp(o - m)
    outc_ref[...] = e / jnp.sum(e, axis=1, keepdims=True)


def kernel(features, energy, isTrack, track_pt, eta, phi, isMuon, layer,
           incidence_val, W1p, b1p, W2p, b2p, W3p, b3p, W1c, b1c, W2c, b2c,
           W3c, b3c, edge_src, edge_dst):
    E = incidence_val.shape[0]
    BSN = energy.shape[0]
    BSP, DIM = features.shape
    P = E // BSN
    BS = BSP // P
    N = BSN // BS

    mesh = plsc.VectorSubcoreMesh(core_axis_name="c", subcore_axis_name="s")
    sc_agg = functools.partial(
        pl.kernel,
        out_type=jax.ShapeDtypeStruct((BS, 9, LP), jnp.float32),
        mesh=mesh,
        scratch_types=[
            pltpu.VMEM((N * P + _L,), jnp.float32),
            pltpu.VMEM((7 * N,), jnp.float32),
            pltpu.VMEM((9 * N,), jnp.float32),
            pltpu.VMEM((9, LP), jnp.float32),
        ],
    )(functools.partial(_sc_agg_body, N, P))
    s_agg = sc_agg(energy, isTrack, track_pt, eta, phi, isMuon, layer,
                   incidence_val)

    row2 = lambda a: a.reshape(1, -1)
    hargs = [s_agg, features,
             W1p[:DIM], W1p[DIM:], row2(b1p), W2p, row2(b2p), W3p, row2(b3p),
             W1c[:DIM], W1c[DIM:], row2(b1c), W2c, row2(b2c), W3c, row2(b3c)]
    outp, outc, topo = pl.pallas_call(
        _heads_kernel,
        in_specs=[pl.BlockSpec(a.shape, lambda nd=a.ndim: (0,) * nd)
                  for a in hargs],
        out_specs=[
            pl.BlockSpec((BSP, 3), lambda: (0, 0)),
            pl.BlockSpec((BSP, 6), lambda: (0, 0)),
            pl.BlockSpec((BSP, 4), lambda: (0, 0)),
        ],
        out_shape=[
            jax.ShapeDtypeStruct((BSP, 3), jnp.float32),
            jax.ShapeDtypeStruct((BSP, 6), jnp.float32),
            jax.ShapeDtypeStruct((BSP, 4), jnp.float32),
        ],
    )(*hargs)

    return (outp.reshape(BS, P, 3), outc.reshape(BS, P, 6), topo)
